# Initial kernel scaffold; baseline (speedup 1.0000x reference)
import jax
import jax.numpy as jnp
from jax.experimental import pallas as pl


def kernel(x, edge_index, current_vertex, Wl1, Wr1, att1, b1, Wl2, Wr2, att2, b2, Wl3, Wr3, att3, b3, W4, b4, W5, b5, W6, b6):
    raise NotImplementedError("write your pallas kernel here")


# speedup vs baseline: 2.9601x; 2.9601x over previous
import functools

import jax
import jax.numpy as jnp
from jax import lax
from jax.experimental import pallas as pl
from jax.experimental.pallas import tpu as pltpu
from jax.experimental.pallas import tpu_sc as plsc

N = 10000
E_RAW = 320000
ET = E_RAW + N
DIN = 128
H = 512
NPAD = 10240
NC, NS, LANES = 2, 16, 16
NW = NC * NS
T_TILE = 10496
EPAD = NW * T_TILE
GB = 128
DUMMY = N
CW = 128


def _sc_mesh():
    return plsc.VectorSubcoreMesh(
        core_axis_name="c", subcore_axis_name="s",
        num_cores=NC, num_subcores=NS)


def _worker(cid, sid):
    return sid * NC + cid


def _sc_gather(table, idx):
    @functools.partial(
        pl.kernel,
        out_type=jax.ShapeDtypeStruct((EPAD, H), jnp.float32),
        mesh=_sc_mesh(),
        scratch_types=[
            pltpu.VMEM((GB,), jnp.int32),
            pltpu.VMEM((GB, H), jnp.float32),
            pltpu.SemaphoreType.DMA,
        ],
    )
    def k(table_hbm, idx_hbm, out_hbm, idx_v, rows_v, sem):
        base0 = _worker(lax.axis_index("c"), lax.axis_index("s")) * T_TILE

        def body(i, carry):
            base = base0 + i * GB
            pltpu.sync_copy(idx_hbm.at[pl.ds(base, GB)], idx_v)
            pltpu.async_copy(table_hbm.at[idx_v], rows_v, sem).wait()
            pltpu.sync_copy(rows_v, out_hbm.at[pl.ds(base, GB)])
            return carry

        lax.fori_loop(0, T_TILE // GB, body, 0)

    return k(table, idx)


def _sc_scatter_add(vals, idx, cw=CW):
    rps = NPAD // NS

    @functools.partial(
        pl.kernel,
        out_type=jax.ShapeDtypeStruct((NC * NPAD, cw), jnp.float32),
        mesh=_sc_mesh(),
        scratch_types=[
            pltpu.VMEM((GB,), jnp.int32),
            pltpu.VMEM((GB, cw), jnp.float32),
            pltpu.VMEM_SHARED((NPAD, cw), jnp.float32),
        ],
    )
    def k(vals_hbm, idx_hbm, out_hbm, idx_v, rows_v, acc_sp):
        cid = lax.axis_index("c")
        sid = lax.axis_index("s")
        base0 = _worker(cid, sid) * T_TILE

        def zrow(i, carry):
            for jj in range(cw // LANES):
                rows_v[i, pl.ds(jj * LANES, LANES)] = jnp.zeros(
                    (LANES,), jnp.float32)
            return carry

        lax.fori_loop(0, GB, zrow, 0)

        def zcp(i, carry):
            pltpu.sync_copy(
                rows_v, acc_sp.at[pl.ds(sid * rps + i * GB, GB)])
            return carry

        lax.fori_loop(0, rps // GB, zcp, 0)
        plsc.subcore_barrier()

        def body(i, carry):
            base = base0 + i * GB
            pltpu.sync_copy(idx_hbm.at[pl.ds(base, GB)], idx_v)
            pltpu.sync_copy(vals_hbm.at[pl.ds(base, GB)], rows_v)
            pltpu.sync_copy(rows_v, acc_sp.at[idx_v], add=True)
            return carry

        lax.fori_loop(0, T_TILE // GB, body, 0)
        plsc.subcore_barrier()

        def dump(i, carry):
            r = sid * rps + i * GB
            pltpu.sync_copy(acc_sp.at[pl.ds(r, GB)], rows_v)
            pltpu.sync_copy(rows_v, out_hbm.at[pl.ds(cid * NPAD + r, GB)])
            return carry

        lax.fori_loop(0, rps // GB, dump, 0)

    return k(vals, idx)


def _tc_flags(src2d, cv_row):
    eg = src2d.shape[0]

    def body(src_ref, cv_ref, f_ref):
        sv = src_ref[...]
        row = lax.broadcasted_iota(jnp.int32, (eg, 128), 0)
        pos = row * 128 + lax.broadcasted_iota(jnp.int32, (eg, 128), 1)
        f_ref[...] = jnp.where((sv == cv_ref[...]) & (pos < E_RAW),
                               jnp.float32(1.0), jnp.float32(0.0))

    return pl.pallas_call(
        body,
        out_shape=jax.ShapeDtypeStruct((eg, 128), jnp.float32),
    )(src2d, cv_row)


def _tc_matmul2(a, wl, wr):
    m, kdim = a.shape
    bm = 512

    def body(a_ref, wl_ref, wr_ref, xl_ref, xr_ref):
        av = a_ref[...]
        xl_ref[...] = jnp.dot(av, wl_ref[...],
                              preferred_element_type=jnp.float32)
        xr_ref[...] = jnp.dot(av, wr_ref[...],
                              preferred_element_type=jnp.float32)

    return pl.pallas_call(
        body,
        grid=(m // bm,),
        in_specs=[
            pl.BlockSpec((bm, kdim), lambda i: (i, 0)),
            pl.BlockSpec((kdim, H), lambda i: (0, 0)),
            pl.BlockSpec((kdim, H), lambda i: (0, 0)),
        ],
        out_specs=[
            pl.BlockSpec((bm, H), lambda i: (i, 0)),
            pl.BlockSpec((bm, H), lambda i: (i, 0)),
        ],
        out_shape=[jax.ShapeDtypeStruct((m, H), jnp.float32)] * 2,
    )(a, wl, wr)


def _tc_score(gl, gr, att):
    bm = 512

    def body(gl_ref, gr_ref, att_ref, w_ref, p0, p1, p2, p3):
        glv = gl_ref[...]
        z = glv + gr_ref[...]
        l = jnp.maximum(z, 0.2 * z)
        s = jnp.sum(l * att_ref[...], axis=1, keepdims=True)
        w = jnp.exp(s)
        w_ref[...] = jnp.broadcast_to(w, (bm, CW))
        for c, pr in enumerate((p0, p1, p2, p3)):
            pr[...] = w * glv[:, c * CW:(c + 1) * CW]

    return pl.pallas_call(
        body,
        grid=(EPAD // bm,),
        in_specs=[
            pl.BlockSpec((bm, H), lambda i: (i, 0)),
            pl.BlockSpec((bm, H), lambda i: (i, 0)),
            pl.BlockSpec((1, H), lambda i: (0, 0)),
        ],
        out_specs=[pl.BlockSpec((bm, CW), lambda i: (i, 0))] * 5,
        out_shape=[jax.ShapeDtypeStruct((EPAD, CW), jnp.float32)] * 5,
    )(gl, gr, att.reshape(1, H))


def _tc_finish(na, nb, d0, d1, b):
    bm = 1024

    def body(na_ref, nb_ref, d0_ref, d1_ref, b_ref, h_ref):
        den = d0_ref[...][:, 0:1] + d1_ref[...][:, 0:1] + 1e-16
        h = jnp.tanh((na_ref[...] + nb_ref[...]) / den + b_ref[...])
        rid = (lax.broadcasted_iota(jnp.int32, (bm, 1), 0)
               + pl.program_id(0) * bm)
        h_ref[...] = jnp.where(rid < N, h, 0.0)

    return pl.pallas_call(
        body,
        grid=(NPAD // bm,),
        in_specs=[
            pl.BlockSpec((bm, H), lambda i: (i, 0)),
            pl.BlockSpec((bm, H), lambda i: (i, 0)),
            pl.BlockSpec((bm, CW), lambda i: (i, 0)),
            pl.BlockSpec((bm, CW), lambda i: (i, 0)),
            pl.BlockSpec((1, H), lambda i: (0, 0)),
        ],
        out_specs=pl.BlockSpec((bm, H), lambda i: (i, 0)),
        out_shape=jax.ShapeDtypeStruct((NPAD, H), jnp.float32),
    )(na, nb, d0, d1, b.reshape(1, H))


def _tc_head(h, w4, b4, w5, b5, w6, b6):
    bm = 1024

    def body(h_ref, w4_ref, b4_ref, w5_ref, b5_ref, w6_ref, b6_ref, o_ref):
        t = jnp.dot(h_ref[...], w4_ref[...],
                    preferred_element_type=jnp.float32) + b4_ref[...]
        t = jnp.maximum(t, 0.01 * t)
        t = jnp.dot(t, w5_ref[...],
                    preferred_element_type=jnp.float32) + b5_ref[...]
        t = jnp.maximum(t, 0.01 * t)
        o_ref[...] = jnp.dot(t, w6_ref[...],
                             preferred_element_type=jnp.float32) + b6_ref[...]

    return pl.pallas_call(
        body,
        grid=(NPAD // bm,),
        in_specs=[
            pl.BlockSpec((bm, H), lambda i: (i, 0)),
            pl.BlockSpec((H, H), lambda i: (0, 0)),
            pl.BlockSpec((1, H), lambda i: (0, 0)),
            pl.BlockSpec((H, H), lambda i: (0, 0)),
            pl.BlockSpec((1, H), lambda i: (0, 0)),
            pl.BlockSpec((H, 1), lambda i: (0, 0)),
            pl.BlockSpec((1, 1), lambda i: (0, 0)),
        ],
        out_specs=pl.BlockSpec((bm, 1), lambda i: (i, 0)),
        out_shape=jax.ShapeDtypeStruct((NPAD, 1), jnp.float32),
    )(h, w4, b4.reshape(1, H), w5, b5.reshape(1, H),
      w6.reshape(H, 1), b6.reshape(1, 1))


def _tc_softmax(logits, hits):
    def body(lg_ref, h_ref, o_ref):
        lg = lg_ref[...]
        hv = h_ref[...]
        nbr = hv[:NPAD, 0:1] + hv[NPAD:, 0:1]
        rows = lax.broadcasted_iota(jnp.int32, (NPAD, 1), 0)
        valid = rows < N
        lm = jnp.where(valid & (nbr > 0), lg,
                       jnp.where(valid, jnp.float32(-1e9),
                                 jnp.float32(-1e30)))
        m = jnp.max(lm)
        p = jnp.exp(lm - m)
        o_ref[...] = p / jnp.sum(p)

    return pl.pallas_call(
        body,
        out_shape=jax.ShapeDtypeStruct((NPAD, 1), jnp.float32),
    )(logits, hits)


def kernel(x, edge_index, current_vertex,
           Wl1, Wr1, att1, b1, Wl2, Wr2, att2, b2, Wl3, Wr3, att3, b3,
           W4, b4, W5, b5, W6, b6):
    f32 = jnp.float32
    xp = jnp.zeros((NPAD, DIN), f32).at[:N, :].set(x)
    loops = jnp.arange(N, dtype=jnp.int32)
    padi = jnp.full((EPAD - ET,), DUMMY, jnp.int32)
    src = jnp.concatenate([edge_index[0].astype(jnp.int32), loops, padi])
    dst = jnp.concatenate([edge_index[1].astype(jnp.int32), loops, padi])
    cv_row = jnp.full((1, 128), current_vertex, jnp.int32)

    flags = _tc_flags(src.reshape(EPAD // 128, 128), cv_row)
    flag128 = jnp.broadcast_to(flags.reshape(EPAD, 1), (EPAD, CW))
    hits = _sc_scatter_add(flag128, dst)

    h = xp
    for (Wl, Wr, att, b) in ((Wl1, Wr1, att1, b1),
                             (Wl2, Wr2, att2, b2),
                             (Wl3, Wr3, att3, b3)):
        xl, xr = _tc_matmul2(h, Wl, Wr)
        gl = _sc_gather(xl, src)
        gr = _sc_gather(xr, dst)
        w128, p0, p1, p2, p3 = _tc_score(gl, gr, att)
        dsum = _sc_scatter_add(w128, dst)
        nums = [_sc_scatter_add(p, dst) for p in (p0, p1, p2, p3)]
        na = jnp.concatenate([n[:NPAD] for n in nums], axis=1)
        nb = jnp.concatenate([n[NPAD:] for n in nums], axis=1)
        h = _tc_finish(na, nb, dsum[:NPAD], dsum[NPAD:], b)

    logits = _tc_head(h, W4, b4, W5, b5, W6, b6)
    probs = _tc_softmax(logits, hits)
    return probs[:N, 0]


# speedup vs baseline: 3.2959x; 1.1134x over previous
import functools

import jax
import jax.numpy as jnp
from jax import lax
from jax.experimental import pallas as pl
from jax.experimental.pallas import tpu as pltpu
from jax.experimental.pallas import tpu_sc as plsc

N = 10000
E_RAW = 320000
ET = E_RAW + N
DIN = 128
H = 512
NPAD = 10240
NC, NS, LANES = 2, 16, 16
NW = NC * NS
T_TILE = 10496
EPAD = NW * T_TILE
GB = 128
DUMMY = N
CW = 128


def _sc_mesh():
    return plsc.VectorSubcoreMesh(
        core_axis_name="c", subcore_axis_name="s",
        num_cores=NC, num_subcores=NS)


def _worker(cid, sid):
    return sid * NC + cid


GBH = 64


def _sc_gather(table, idx):
    @functools.partial(
        pl.kernel,
        out_type=jax.ShapeDtypeStruct((EPAD, H), jnp.float32),
        mesh=_sc_mesh(),
        scratch_types=[
            pltpu.VMEM((GBH,), jnp.int32),
            pltpu.VMEM((GBH,), jnp.int32),
            pltpu.VMEM((GBH, H), jnp.float32),
            pltpu.VMEM((GBH, H), jnp.float32),
            pltpu.SemaphoreType.DMA,
            pltpu.SemaphoreType.DMA,
            pltpu.SemaphoreType.DMA,
            pltpu.SemaphoreType.DMA,
        ],
    )
    def k(table_hbm, idx_hbm, out_hbm,
          idx_a, idx_b, rows_a, rows_b, sga, sgb, swa, swb):
        base0 = _worker(lax.axis_index("c"), lax.axis_index("s")) * T_TILE

        def body(i, carry):
            ba = base0 + (2 * i) * GBH
            bb = ba + GBH
            pltpu.sync_copy(idx_hbm.at[pl.ds(ba, GBH)], idx_a)
            ga = pltpu.async_copy(table_hbm.at[idx_a], rows_a, sga)
            pltpu.sync_copy(idx_hbm.at[pl.ds(bb, GBH)], idx_b)
            gb = pltpu.async_copy(table_hbm.at[idx_b], rows_b, sgb)
            ga.wait()
            wa = pltpu.async_copy(rows_a, out_hbm.at[pl.ds(ba, GBH)], swa)
            gb.wait()
            wb = pltpu.async_copy(rows_b, out_hbm.at[pl.ds(bb, GBH)], swb)
            wa.wait()
            wb.wait()
            return carry

        lax.fori_loop(0, T_TILE // (2 * GBH), body, 0)

    return k(table, idx)


def _sc_scatter_add(vals, idx, cw=CW):
    rps = NPAD // NS

    @functools.partial(
        pl.kernel,
        out_type=jax.ShapeDtypeStruct((NC * NPAD, cw), jnp.float32),
        mesh=_sc_mesh(),
        scratch_types=[
            pltpu.VMEM((GB,), jnp.int32),
            pltpu.VMEM((GB,), jnp.int32),
            pltpu.VMEM((GB, cw), jnp.float32),
            pltpu.VMEM((GB, cw), jnp.float32),
            pltpu.SemaphoreType.DMA,
            pltpu.SemaphoreType.DMA,
            pltpu.SemaphoreType.DMA,
            pltpu.SemaphoreType.DMA,
            pltpu.VMEM_SHARED((NPAD, cw), jnp.float32),
        ],
    )
    def k(vals_hbm, idx_hbm, out_hbm,
          idx_a, idx_b, rows_a, rows_b, sia, sib, sra, srb, acc_sp):
        cid = lax.axis_index("c")
        sid = lax.axis_index("s")
        base0 = _worker(cid, sid) * T_TILE

        def zrow(i, carry):
            for jj in range(cw // LANES):
                rows_a[i, pl.ds(jj * LANES, LANES)] = jnp.zeros(
                    (LANES,), jnp.float32)
            return carry

        lax.fori_loop(0, GB, zrow, 0)

        def zcp(i, carry):
            pltpu.sync_copy(
                rows_a, acc_sp.at[pl.ds(sid * rps + i * GB, GB)])
            return carry

        lax.fori_loop(0, rps // GB, zcp, 0)
        plsc.subcore_barrier()

        def body(i, carry):
            ba = base0 + (2 * i) * GB
            bb = ba + GB
            ia = pltpu.async_copy(idx_hbm.at[pl.ds(ba, GB)], idx_a, sia)
            ra = pltpu.async_copy(vals_hbm.at[pl.ds(ba, GB)], rows_a, sra)
            ib = pltpu.async_copy(idx_hbm.at[pl.ds(bb, GB)], idx_b, sib)
            rb = pltpu.async_copy(vals_hbm.at[pl.ds(bb, GB)], rows_b, srb)
            ia.wait()
            ra.wait()
            pltpu.sync_copy(rows_a, acc_sp.at[idx_a], add=True)
            ib.wait()
            rb.wait()
            pltpu.sync_copy(rows_b, acc_sp.at[idx_b], add=True)
            return carry

        lax.fori_loop(0, T_TILE // (2 * GB), body, 0)
        plsc.subcore_barrier()

        def dump(i, carry):
            r = sid * rps + i * GB
            pltpu.sync_copy(acc_sp.at[pl.ds(r, GB)], rows_a)
            pltpu.sync_copy(rows_a, out_hbm.at[pl.ds(cid * NPAD + r, GB)])
            return carry

        lax.fori_loop(0, rps // GB, dump, 0)

    return k(vals, idx)


def _tc_flags(src2d, cv_row):
    eg = src2d.shape[0]

    def body(src_ref, cv_ref, f_ref):
        sv = src_ref[...]
        row = lax.broadcasted_iota(jnp.int32, (eg, 128), 0)
        pos = row * 128 + lax.broadcasted_iota(jnp.int32, (eg, 128), 1)
        f_ref[...] = jnp.where((sv == cv_ref[...]) & (pos < E_RAW),
                               jnp.float32(1.0), jnp.float32(0.0))

    return pl.pallas_call(
        body,
        out_shape=jax.ShapeDtypeStruct((eg, 128), jnp.float32),
    )(src2d, cv_row)


def _tc_matmul2(a, wl, wr):
    m, kdim = a.shape
    bm = 512

    def body(a_ref, wl_ref, wr_ref, xl_ref, xr_ref):
        av = a_ref[...]
        xl_ref[...] = jnp.dot(av, wl_ref[...],
                              preferred_element_type=jnp.float32)
        xr_ref[...] = jnp.dot(av, wr_ref[...],
                              preferred_element_type=jnp.float32)

    return pl.pallas_call(
        body,
        grid=(m // bm,),
        in_specs=[
            pl.BlockSpec((bm, kdim), lambda i: (i, 0)),
            pl.BlockSpec((kdim, H), lambda i: (0, 0)),
            pl.BlockSpec((kdim, H), lambda i: (0, 0)),
        ],
        out_specs=[
            pl.BlockSpec((bm, H), lambda i: (i, 0)),
            pl.BlockSpec((bm, H), lambda i: (i, 0)),
        ],
        out_shape=[jax.ShapeDtypeStruct((m, H), jnp.float32)] * 2,
    )(a, wl, wr)


def _tc_score(gl, gr, att):
    bm = 512

    def body(gl_ref, gr_ref, att_ref, w_ref, p0, p1, p2, p3):
        glv = gl_ref[...]
        z = glv + gr_ref[...]
        l = jnp.maximum(z, 0.2 * z)
        s = jnp.sum(l * att_ref[...], axis=1, keepdims=True)
        w = jnp.exp(s)
        w_ref[...] = jnp.broadcast_to(w, (bm, CW))
        for c, pr in enumerate((p0, p1, p2, p3)):
            pr[...] = w * glv[:, c * CW:(c + 1) * CW]

    return pl.pallas_call(
        body,
        grid=(EPAD // bm,),
        in_specs=[
            pl.BlockSpec((bm, H), lambda i: (i, 0)),
            pl.BlockSpec((bm, H), lambda i: (i, 0)),
            pl.BlockSpec((1, H), lambda i: (0, 0)),
        ],
        out_specs=[pl.BlockSpec((bm, CW), lambda i: (i, 0))] * 5,
        out_shape=[jax.ShapeDtypeStruct((EPAD, CW), jnp.float32)] * 5,
    )(gl, gr, att.reshape(1, H))


def _tc_finish(na, nb, d0, d1, b):
    bm = 1024

    def body(na_ref, nb_ref, d0_ref, d1_ref, b_ref, h_ref):
        den = d0_ref[...][:, 0:1] + d1_ref[...][:, 0:1] + 1e-16
        h = jnp.tanh((na_ref[...] + nb_ref[...]) / den + b_ref[...])
        rid = (lax.broadcasted_iota(jnp.int32, (bm, 1), 0)
               + pl.program_id(0) * bm)
        h_ref[...] = jnp.where(rid < N, h, 0.0)

    return pl.pallas_call(
        body,
        grid=(NPAD // bm,),
        in_specs=[
            pl.BlockSpec((bm, H), lambda i: (i, 0)),
            pl.BlockSpec((bm, H), lambda i: (i, 0)),
            pl.BlockSpec((bm, CW), lambda i: (i, 0)),
            pl.BlockSpec((bm, CW), lambda i: (i, 0)),
            pl.BlockSpec((1, H), lambda i: (0, 0)),
        ],
        out_specs=pl.BlockSpec((bm, H), lambda i: (i, 0)),
        out_shape=jax.ShapeDtypeStruct((NPAD, H), jnp.float32),
    )(na, nb, d0, d1, b.reshape(1, H))


def _tc_head(h, w4, b4, w5, b5, w6, b6):
    bm = 1024

    def body(h_ref, w4_ref, b4_ref, w5_ref, b5_ref, w6_ref, b6_ref, o_ref):
        t = jnp.dot(h_ref[...], w4_ref[...],
                    preferred_element_type=jnp.float32) + b4_ref[...]
        t = jnp.maximum(t, 0.01 * t)
        t = jnp.dot(t, w5_ref[...],
                    preferred_element_type=jnp.float32) + b5_ref[...]
        t = jnp.maximum(t, 0.01 * t)
        o_ref[...] = jnp.dot(t, w6_ref[...],
                             preferred_element_type=jnp.float32) + b6_ref[...]

    return pl.pallas_call(
        body,
        grid=(NPAD // bm,),
        in_specs=[
            pl.BlockSpec((bm, H), lambda i: (i, 0)),
            pl.BlockSpec((H, H), lambda i: (0, 0)),
            pl.BlockSpec((1, H), lambda i: (0, 0)),
            pl.BlockSpec((H, H), lambda i: (0, 0)),
            pl.BlockSpec((1, H), lambda i: (0, 0)),
            pl.BlockSpec((H, 1), lambda i: (0, 0)),
            pl.BlockSpec((1, 1), lambda i: (0, 0)),
        ],
        out_specs=pl.BlockSpec((bm, 1), lambda i: (i, 0)),
        out_shape=jax.ShapeDtypeStruct((NPAD, 1), jnp.float32),
    )(h, w4, b4.reshape(1, H), w5, b5.reshape(1, H),
      w6.reshape(H, 1), b6.reshape(1, 1))


def _tc_softmax(logits, hits):
    def body(lg_ref, h_ref, o_ref):
        lg = lg_ref[...]
        hv = h_ref[...]
        nbr = hv[:NPAD, 0:1] + hv[NPAD:, 0:1]
        rows = lax.broadcasted_iota(jnp.int32, (NPAD, 1), 0)
        valid = rows < N
        lm = jnp.where(valid & (nbr > 0), lg,
                       jnp.where(valid, jnp.float32(-1e9),
                                 jnp.float32(-1e30)))
        m = jnp.max(lm)
        p = jnp.exp(lm - m)
        o_ref[...] = p / jnp.sum(p)

    return pl.pallas_call(
        body,
        out_shape=jax.ShapeDtypeStruct((NPAD, 1), jnp.float32),
    )(logits, hits)


def kernel(x, edge_index, current_vertex,
           Wl1, Wr1, att1, b1, Wl2, Wr2, att2, b2, Wl3, Wr3, att3, b3,
           W4, b4, W5, b5, W6, b6):
    f32 = jnp.float32
    xp = jnp.zeros((NPAD, DIN), f32).at[:N, :].set(x)
    loops = jnp.arange(N, dtype=jnp.int32)
    padi = jnp.full((EPAD - ET,), DUMMY, jnp.int32)
    src = jnp.concatenate([edge_index[0].astype(jnp.int32), loops, padi])
    dst = jnp.concatenate([edge_index[1].astype(jnp.int32), loops, padi])
    cv_row = jnp.full((1, 128), current_vertex, jnp.int32)

    flags = _tc_flags(src.reshape(EPAD // 128, 128), cv_row)
    flag128 = jnp.broadcast_to(flags.reshape(EPAD, 1), (EPAD, CW))
    hits = _sc_scatter_add(flag128, dst)

    h = xp
    for (Wl, Wr, att, b) in ((Wl1, Wr1, att1, b1),
                             (Wl2, Wr2, att2, b2),
                             (Wl3, Wr3, att3, b3)):
        xl, xr = _tc_matmul2(h, Wl, Wr)
        gl = _sc_gather(xl, src)
        gr = _sc_gather(xr, dst)
        w128, p0, p1, p2, p3 = _tc_score(gl, gr, att)
        dsum = _sc_scatter_add(w128, dst)
        nums = [_sc_scatter_add(p, dst) for p in (p0, p1, p2, p3)]
        na = jnp.concatenate([n[:NPAD] for n in nums], axis=1)
        nb = jnp.concatenate([n[NPAD:] for n in nums], axis=1)
        h = _tc_finish(na, nb, dsum[:NPAD], dsum[NPAD:], b)

    logits = _tc_head(h, W4, b4, W5, b5, W6, b6)
    probs = _tc_softmax(logits, hits)
    return probs[:N, 0]


# speedup vs baseline: 3.7254x; 1.1303x over previous
import functools

import jax
import jax.numpy as jnp
from jax import lax
from jax.experimental import pallas as pl
from jax.experimental.pallas import tpu as pltpu
from jax.experimental.pallas import tpu_sc as plsc

N = 10000
E_RAW = 320000
ET = E_RAW + N
DIN = 128
H = 512
NPAD = 10240
NC, NS, LANES = 2, 16, 16
NW = NC * NS
T_TILE = 10496
EPAD = NW * T_TILE
GB = 128
DUMMY = N
CW = 128


def _sc_mesh():
    return plsc.VectorSubcoreMesh(
        core_axis_name="c", subcore_axis_name="s",
        num_cores=NC, num_subcores=NS)


def _worker(cid, sid):
    return sid * NC + cid


GBH = 64


HW = H // 2


def _sc_gather(table, idx):
    @functools.partial(
        pl.kernel,
        out_type=jax.ShapeDtypeStruct((EPAD, HW), jnp.int32),
        mesh=_sc_mesh(),
        scratch_types=[
            pltpu.VMEM((GBH,), jnp.int32),
            pltpu.VMEM((GBH,), jnp.int32),
            pltpu.VMEM((GBH, HW), jnp.int32),
            pltpu.VMEM((GBH, HW), jnp.int32),
            pltpu.SemaphoreType.DMA,
            pltpu.SemaphoreType.DMA,
            pltpu.SemaphoreType.DMA,
            pltpu.SemaphoreType.DMA,
        ],
    )
    def k(table_hbm, idx_hbm, out_hbm,
          idx_a, idx_b, rows_a, rows_b, sga, sgb, swa, swb):
        base0 = _worker(lax.axis_index("c"), lax.axis_index("s")) * T_TILE

        def body(i, carry):
            ba = base0 + (2 * i) * GBH
            bb = ba + GBH
            pltpu.sync_copy(idx_hbm.at[pl.ds(ba, GBH)], idx_a)
            ga = pltpu.async_copy(table_hbm.at[idx_a], rows_a, sga)
            pltpu.sync_copy(idx_hbm.at[pl.ds(bb, GBH)], idx_b)
            gb = pltpu.async_copy(table_hbm.at[idx_b], rows_b, sgb)
            ga.wait()
            wa = pltpu.async_copy(rows_a, out_hbm.at[pl.ds(ba, GBH)], swa)
            gb.wait()
            wb = pltpu.async_copy(rows_b, out_hbm.at[pl.ds(bb, GBH)], swb)
            wa.wait()
            wb.wait()
            return carry

        lax.fori_loop(0, T_TILE // (2 * GBH), body, 0)

    return k(table, idx)


def _sc_scatter_add(vals, idx, cw=CW):
    rps = NPAD // NS

    @functools.partial(
        pl.kernel,
        out_type=jax.ShapeDtypeStruct((NC * NPAD, cw), jnp.float32),
        mesh=_sc_mesh(),
        scratch_types=[
            pltpu.VMEM((GB,), jnp.int32),
            pltpu.VMEM((GB,), jnp.int32),
            pltpu.VMEM((GB, cw), jnp.float32),
            pltpu.VMEM((GB, cw), jnp.float32),
            pltpu.SemaphoreType.DMA,
            pltpu.SemaphoreType.DMA,
            pltpu.SemaphoreType.DMA,
            pltpu.SemaphoreType.DMA,
            pltpu.VMEM_SHARED((NPAD, cw), jnp.float32),
        ],
    )
    def k(vals_hbm, idx_hbm, out_hbm,
          idx_a, idx_b, rows_a, rows_b, sia, sib, sra, srb, acc_sp):
        cid = lax.axis_index("c")
        sid = lax.axis_index("s")
        base0 = _worker(cid, sid) * T_TILE

        def zrow(i, carry):
            for jj in range(cw // LANES):
                rows_a[i, pl.ds(jj * LANES, LANES)] = jnp.zeros(
                    (LANES,), jnp.float32)
            return carry

        lax.fori_loop(0, GB, zrow, 0)

        def zcp(i, carry):
            pltpu.sync_copy(
                rows_a, acc_sp.at[pl.ds(sid * rps + i * GB, GB)])
            return carry

        lax.fori_loop(0, rps // GB, zcp, 0)
        plsc.subcore_barrier()

        def body(i, carry):
            ba = base0 + (2 * i) * GB
            bb = ba + GB
            ia = pltpu.async_copy(idx_hbm.at[pl.ds(ba, GB)], idx_a, sia)
            ra = pltpu.async_copy(vals_hbm.at[pl.ds(ba, GB)], rows_a, sra)
            ib = pltpu.async_copy(idx_hbm.at[pl.ds(bb, GB)], idx_b, sib)
            rb = pltpu.async_copy(vals_hbm.at[pl.ds(bb, GB)], rows_b, srb)
            ia.wait()
            ra.wait()
            pltpu.sync_copy(rows_a, acc_sp.at[idx_a], add=True)
            ib.wait()
            rb.wait()
            pltpu.sync_copy(rows_b, acc_sp.at[idx_b], add=True)
            return carry

        lax.fori_loop(0, T_TILE // (2 * GB), body, 0)
        plsc.subcore_barrier()

        def dump(i, carry):
            r = sid * rps + i * GB
            pltpu.sync_copy(acc_sp.at[pl.ds(r, GB)], rows_a)
            pltpu.sync_copy(rows_a, out_hbm.at[pl.ds(cid * NPAD + r, GB)])
            return carry

        lax.fori_loop(0, rps // GB, dump, 0)

    return k(vals, idx)


def _tc_flags(src2d, cv_row):
    eg = src2d.shape[0]

    def body(src_ref, cv_ref, f_ref):
        sv = src_ref[...]
        row = lax.broadcasted_iota(jnp.int32, (eg, 128), 0)
        pos = row * 128 + lax.broadcasted_iota(jnp.int32, (eg, 128), 1)
        f_ref[...] = jnp.where((sv == cv_ref[...]) & (pos < E_RAW),
                               jnp.float32(1.0), jnp.float32(0.0))

    return pl.pallas_call(
        body,
        out_shape=jax.ShapeDtypeStruct((eg, 128), jnp.float32),
    )(src2d, cv_row)


def _tc_matmul2(a, wl, wr):
    m, kdim = a.shape
    bm = 512

    def body(a_ref, wl_ref, wr_ref, xl_ref, xr_ref):
        av = a_ref[...]
        xl_ref[...] = jnp.dot(av, wl_ref[...],
                              preferred_element_type=jnp.float32
                              ).astype(jnp.bfloat16)
        xr_ref[...] = jnp.dot(av, wr_ref[...],
                              preferred_element_type=jnp.float32
                              ).astype(jnp.bfloat16)

    return pl.pallas_call(
        body,
        grid=(m // bm,),
        in_specs=[
            pl.BlockSpec((bm, kdim), lambda i: (i, 0)),
            pl.BlockSpec((kdim, H), lambda i: (0, 0)),
            pl.BlockSpec((kdim, H), lambda i: (0, 0)),
        ],
        out_specs=[
            pl.BlockSpec((bm, H), lambda i: (i, 0)),
            pl.BlockSpec((bm, H), lambda i: (i, 0)),
        ],
        out_shape=[jax.ShapeDtypeStruct((m, H), jnp.bfloat16)] * 2,
    )(a, wl, wr)


def _tc_score(gl, gr, att_d):
    bm = 512

    def body(gl_ref, gr_ref, att_ref, w_ref, p0, p1, p2, p3):
        a = gl_ref[...]
        b = gr_ref[...]
        glo = pltpu.bitcast(a << 16, jnp.float32)
        ghi = pltpu.bitcast(a & -65536, jnp.float32)
        rlo = pltpu.bitcast(b << 16, jnp.float32)
        rhi = pltpu.bitcast(b & -65536, jnp.float32)
        atv = att_ref[...]
        zlo = glo + rlo
        zhi = ghi + rhi
        llo = jnp.maximum(zlo, 0.2 * zlo)
        lhi = jnp.maximum(zhi, 0.2 * zhi)
        s = jnp.sum(llo * atv[:, :HW] + lhi * atv[:, HW:],
                    axis=1, keepdims=True)
        w = jnp.exp(s)
        w_ref[...] = jnp.broadcast_to(w, (bm, CW))
        p0[...] = w * glo[:, :CW]
        p1[...] = w * glo[:, CW:]
        p2[...] = w * ghi[:, :CW]
        p3[...] = w * ghi[:, CW:]

    return pl.pallas_call(
        body,
        grid=(EPAD // bm,),
        in_specs=[
            pl.BlockSpec((bm, HW), lambda i: (i, 0)),
            pl.BlockSpec((bm, HW), lambda i: (i, 0)),
            pl.BlockSpec((1, H), lambda i: (0, 0)),
        ],
        out_specs=[pl.BlockSpec((bm, CW), lambda i: (i, 0))] * 5,
        out_shape=[jax.ShapeDtypeStruct((EPAD, CW), jnp.float32)] * 5,
    )(gl, gr, att_d.reshape(1, H))


def _tc_finish(na, nb, d0, d1, b):
    bm = 1024

    def body(na_ref, nb_ref, d0_ref, d1_ref, b_ref, h_ref):
        den = d0_ref[...][:, 0:1] + d1_ref[...][:, 0:1] + 1e-16
        h = jnp.tanh((na_ref[...] + nb_ref[...]) / den + b_ref[...])
        rid = (lax.broadcasted_iota(jnp.int32, (bm, 1), 0)
               + pl.program_id(0) * bm)
        h_ref[...] = jnp.where(rid < N, h, 0.0)

    return pl.pallas_call(
        body,
        grid=(NPAD // bm,),
        in_specs=[
            pl.BlockSpec((bm, H), lambda i: (i, 0)),
            pl.BlockSpec((bm, H), lambda i: (i, 0)),
            pl.BlockSpec((bm, CW), lambda i: (i, 0)),
            pl.BlockSpec((bm, CW), lambda i: (i, 0)),
            pl.BlockSpec((1, H), lambda i: (0, 0)),
        ],
        out_specs=pl.BlockSpec((bm, H), lambda i: (i, 0)),
        out_shape=jax.ShapeDtypeStruct((NPAD, H), jnp.float32),
    )(na, nb, d0, d1, b.reshape(1, H))


def _tc_head(h, w4, b4, w5, b5, w6, b6):
    bm = 1024

    def body(h_ref, w4_ref, b4_ref, w5_ref, b5_ref, w6_ref, b6_ref, o_ref):
        t = jnp.dot(h_ref[...], w4_ref[...],
                    preferred_element_type=jnp.float32) + b4_ref[...]
        t = jnp.maximum(t, 0.01 * t)
        t = jnp.dot(t, w5_ref[...],
                    preferred_element_type=jnp.float32) + b5_ref[...]
        t = jnp.maximum(t, 0.01 * t)
        o_ref[...] = jnp.dot(t, w6_ref[...],
                             preferred_element_type=jnp.float32) + b6_ref[...]

    return pl.pallas_call(
        body,
        grid=(NPAD // bm,),
        in_specs=[
            pl.BlockSpec((bm, H), lambda i: (i, 0)),
            pl.BlockSpec((H, H), lambda i: (0, 0)),
            pl.BlockSpec((1, H), lambda i: (0, 0)),
            pl.BlockSpec((H, H), lambda i: (0, 0)),
            pl.BlockSpec((1, H), lambda i: (0, 0)),
            pl.BlockSpec((H, 1), lambda i: (0, 0)),
            pl.BlockSpec((1, 1), lambda i: (0, 0)),
        ],
        out_specs=pl.BlockSpec((bm, 1), lambda i: (i, 0)),
        out_shape=jax.ShapeDtypeStruct((NPAD, 1), jnp.float32),
    )(h, w4, b4.reshape(1, H), w5, b5.reshape(1, H),
      w6.reshape(H, 1), b6.reshape(1, 1))


def _tc_softmax(logits, hits):
    def body(lg_ref, h_ref, o_ref):
        lg = lg_ref[...]
        hv = h_ref[...]
        nbr = hv[:NPAD, 0:1] + hv[NPAD:, 0:1]
        rows = lax.broadcasted_iota(jnp.int32, (NPAD, 1), 0)
        valid = rows < N
        lm = jnp.where(valid & (nbr > 0), lg,
                       jnp.where(valid, jnp.float32(-1e9),
                                 jnp.float32(-1e30)))
        m = jnp.max(lm)
        p = jnp.exp(lm - m)
        o_ref[...] = p / jnp.sum(p)

    return pl.pallas_call(
        body,
        out_shape=jax.ShapeDtypeStruct((NPAD, 1), jnp.float32),
    )(logits, hits)


def kernel(x, edge_index, current_vertex,
           Wl1, Wr1, att1, b1, Wl2, Wr2, att2, b2, Wl3, Wr3, att3, b3,
           W4, b4, W5, b5, W6, b6):
    f32 = jnp.float32
    xp = jnp.zeros((NPAD, DIN), f32).at[:N, :].set(x)
    loops = jnp.arange(N, dtype=jnp.int32)
    padi = jnp.full((EPAD - ET,), DUMMY, jnp.int32)
    src = jnp.concatenate([edge_index[0].astype(jnp.int32), loops, padi])
    dst = jnp.concatenate([edge_index[1].astype(jnp.int32), loops, padi])
    cv_row = jnp.full((1, 128), current_vertex, jnp.int32)

    flags = _tc_flags(src.reshape(EPAD // 128, 128), cv_row)
    flag128 = jnp.broadcast_to(flags.reshape(EPAD, 1), (EPAD, CW))
    hits = _sc_scatter_add(flag128, dst)

    perm = jnp.concatenate([jnp.arange(0, H, 2), jnp.arange(1, H, 2)])

    def pack(t):
        return lax.bitcast_convert_type(
            t.reshape(NPAD, HW, 2), jnp.int32)

    h = xp
    for li, (Wl, Wr, att, b) in enumerate((
            (Wl1, Wr1, att1, b1),
            (Wl2, Wr2, att2, b2),
            (Wl3, Wr3, att3, b3))):
        if li > 0:
            Wl, Wr = Wl[perm, :], Wr[perm, :]
        att_d = jnp.concatenate([att[0::2], att[1::2]])
        xl, xr = _tc_matmul2(h, Wl, Wr)
        gl = _sc_gather(pack(xl), src)
        gr = _sc_gather(pack(xr), dst)
        w128, p0, p1, p2, p3 = _tc_score(gl, gr, att_d)
        dsum = _sc_scatter_add(w128, dst)
        nums = [_sc_scatter_add(p, dst) for p in (p0, p1, p2, p3)]
        na = jnp.concatenate([n[:NPAD] for n in nums], axis=1)
        nb = jnp.concatenate([n[NPAD:] for n in nums], axis=1)
        h = _tc_finish(na, nb, dsum[:NPAD], dsum[NPAD:], b[perm])

    logits = _tc_head(h, W4[perm, :], b4, W5, b5, W6, b6)
    probs = _tc_softmax(logits, hits)
    return probs[:N, 0]


# speedup vs baseline: 3.8228x; 1.0262x over previous
import functools

import jax
import jax.numpy as jnp
from jax import lax
from jax.experimental import pallas as pl
from jax.experimental.pallas import tpu as pltpu
from jax.experimental.pallas import tpu_sc as plsc

N = 10000
E_RAW = 320000
ET = E_RAW + N
DIN = 128
H = 512
NPAD = 10240
NC, NS, LANES = 2, 16, 16
NW = NC * NS
T_TILE = 10496
EPAD = NW * T_TILE
GB = 128
DUMMY = N
CW = 128


def _sc_mesh():
    return plsc.VectorSubcoreMesh(
        core_axis_name="c", subcore_axis_name="s",
        num_cores=NC, num_subcores=NS)


def _worker(cid, sid):
    return sid * NC + cid


GBH = 128


HW = H // 2


def _sc_gather(table, idx):
    @functools.partial(
        pl.kernel,
        out_type=jax.ShapeDtypeStruct((EPAD, HW), jnp.int32),
        mesh=_sc_mesh(),
        scratch_types=[
            pltpu.VMEM((GBH,), jnp.int32),
            pltpu.VMEM((GBH,), jnp.int32),
            pltpu.VMEM((GBH, HW), jnp.int32),
            pltpu.VMEM((GBH, HW), jnp.int32),
            pltpu.SemaphoreType.DMA,
            pltpu.SemaphoreType.DMA,
            pltpu.SemaphoreType.DMA,
            pltpu.SemaphoreType.DMA,
        ],
    )
    def k(table_hbm, idx_hbm, out_hbm,
          idx_a, idx_b, rows_a, rows_b, sga, sgb, swa, swb):
        base0 = _worker(lax.axis_index("c"), lax.axis_index("s")) * T_TILE

        def body(i, carry):
            ba = base0 + (2 * i) * GBH
            bb = ba + GBH
            pltpu.sync_copy(idx_hbm.at[pl.ds(ba, GBH)], idx_a)
            ga = pltpu.async_copy(table_hbm.at[idx_a], rows_a, sga)
            pltpu.sync_copy(idx_hbm.at[pl.ds(bb, GBH)], idx_b)
            gb = pltpu.async_copy(table_hbm.at[idx_b], rows_b, sgb)
            ga.wait()
            wa = pltpu.async_copy(rows_a, out_hbm.at[pl.ds(ba, GBH)], swa)
            gb.wait()
            wb = pltpu.async_copy(rows_b, out_hbm.at[pl.ds(bb, GBH)], swb)
            wa.wait()
            wb.wait()
            return carry

        lax.fori_loop(0, T_TILE // (2 * GBH), body, 0)

    return k(table, idx)


def _sc_scatter_add(vals, idx, cw=CW):
    rps = NPAD // NS
    GS = 128
    GZ = 128
    nmain = T_TILE // (2 * GS)

    @functools.partial(
        pl.kernel,
        out_type=jax.ShapeDtypeStruct((NC * NPAD, cw), jnp.float32),
        mesh=_sc_mesh(),
        scratch_types=[
            pltpu.VMEM((GS,), jnp.int32),
            pltpu.VMEM((GS,), jnp.int32),
            pltpu.VMEM((GS, cw), jnp.float32),
            pltpu.VMEM((GS, cw), jnp.float32),
            pltpu.SemaphoreType.DMA,
            pltpu.SemaphoreType.DMA,
            pltpu.SemaphoreType.DMA,
            pltpu.SemaphoreType.DMA,
            pltpu.SemaphoreType.DMA,
            pltpu.SemaphoreType.DMA,
            pltpu.VMEM_SHARED((NPAD, cw), jnp.float32),
        ],
    )
    def k(vals_hbm, idx_hbm, out_hbm,
          idx_a, idx_b, rows_a, rows_b, sia, sib, sra, srb, saa, sab,
          acc_sp):
        cid = lax.axis_index("c")
        sid = lax.axis_index("s")
        base0 = _worker(cid, sid) * T_TILE

        def zrow(i, carry):
            for jj in range(cw // LANES):
                rows_a[i, pl.ds(jj * LANES, LANES)] = jnp.zeros(
                    (LANES,), jnp.float32)
            return carry

        lax.fori_loop(0, GZ, zrow, 0)

        def zcp(i, carry):
            pltpu.sync_copy(
                rows_a.at[pl.ds(0, GZ)],
                acc_sp.at[pl.ds(sid * rps + i * GZ, GZ)])
            return carry

        lax.fori_loop(0, rps // GZ, zcp, 0)
        plsc.subcore_barrier()

        def body(i, carry):
            ba = base0 + (2 * i) * GS
            bb = ba + GS
            ia = pltpu.async_copy(idx_hbm.at[pl.ds(ba, GS)], idx_a, sia)
            ra = pltpu.async_copy(vals_hbm.at[pl.ds(ba, GS)], rows_a, sra)
            ib = pltpu.async_copy(idx_hbm.at[pl.ds(bb, GS)], idx_b, sib)
            rb = pltpu.async_copy(vals_hbm.at[pl.ds(bb, GS)], rows_b, srb)
            ia.wait()
            ra.wait()
            ca = pltpu.async_copy(rows_a, acc_sp.at[idx_a], saa, add=True)
            ib.wait()
            rb.wait()
            cb = pltpu.async_copy(rows_b, acc_sp.at[idx_b], sab, add=True)
            ca.wait()
            cb.wait()
            return carry

        lax.fori_loop(0, nmain, body, 0)
        plsc.subcore_barrier()

        def dump(i, carry):
            r = sid * rps + i * GZ
            pltpu.sync_copy(acc_sp.at[pl.ds(r, GZ)], rows_a.at[pl.ds(0, GZ)])
            pltpu.sync_copy(rows_a.at[pl.ds(0, GZ)],
                            out_hbm.at[pl.ds(cid * NPAD + r, GZ)])
            return carry

        lax.fori_loop(0, rps // GZ, dump, 0)

    return k(vals, idx)


def _tc_flags(src2d, cv_row):
    eg = src2d.shape[0]

    def body(src_ref, cv_ref, f_ref):
        sv = src_ref[...]
        row = lax.broadcasted_iota(jnp.int32, (eg, 128), 0)
        pos = row * 128 + lax.broadcasted_iota(jnp.int32, (eg, 128), 1)
        f_ref[...] = jnp.where((sv == cv_ref[...]) & (pos < E_RAW),
                               jnp.float32(1.0), jnp.float32(0.0))

    return pl.pallas_call(
        body,
        out_shape=jax.ShapeDtypeStruct((eg, 128), jnp.float32),
    )(src2d, cv_row)


def _tc_matmul2(a, wl, wr):
    m, kdim = a.shape
    bm = 512

    def body(a_ref, wl_ref, wr_ref, xl_ref, xr_ref):
        av = a_ref[...]
        xl_ref[...] = jnp.dot(av, wl_ref[...],
                              preferred_element_type=jnp.float32
                              ).astype(jnp.bfloat16)
        xr_ref[...] = jnp.dot(av, wr_ref[...],
                              preferred_element_type=jnp.float32
                              ).astype(jnp.bfloat16)

    return pl.pallas_call(
        body,
        grid=(m // bm,),
        in_specs=[
            pl.BlockSpec((bm, kdim), lambda i: (i, 0)),
            pl.BlockSpec((kdim, H), lambda i: (0, 0)),
            pl.BlockSpec((kdim, H), lambda i: (0, 0)),
        ],
        out_specs=[
            pl.BlockSpec((bm, H), lambda i: (i, 0)),
            pl.BlockSpec((bm, H), lambda i: (i, 0)),
        ],
        out_shape=[jax.ShapeDtypeStruct((m, H), jnp.bfloat16)] * 2,
    )(a, wl, wr)


def _tc_score(gl, gr, att_d):
    bm = 512

    def body(gl_ref, gr_ref, att_ref, w_ref, p0, p1, p2, p3):
        a = gl_ref[...]
        b = gr_ref[...]
        glo = pltpu.bitcast(a << 16, jnp.float32)
        ghi = pltpu.bitcast(a & -65536, jnp.float32)
        rlo = pltpu.bitcast(b << 16, jnp.float32)
        rhi = pltpu.bitcast(b & -65536, jnp.float32)
        atv = att_ref[...]
        zlo = glo + rlo
        zhi = ghi + rhi
        llo = jnp.maximum(zlo, 0.2 * zlo)
        lhi = jnp.maximum(zhi, 0.2 * zhi)
        s = jnp.sum(llo * atv[:, :HW] + lhi * atv[:, HW:],
                    axis=1, keepdims=True)
        w = jnp.exp(s)
        w_ref[...] = jnp.broadcast_to(w, (bm, CW))
        p0[...] = w * glo[:, :CW]
        p1[...] = w * glo[:, CW:]
        p2[...] = w * ghi[:, :CW]
        p3[...] = w * ghi[:, CW:]

    return pl.pallas_call(
        body,
        grid=(EPAD // bm,),
        in_specs=[
            pl.BlockSpec((bm, HW), lambda i: (i, 0)),
            pl.BlockSpec((bm, HW), lambda i: (i, 0)),
            pl.BlockSpec((1, H), lambda i: (0, 0)),
        ],
        out_specs=[pl.BlockSpec((bm, CW), lambda i: (i, 0))] * 5,
        out_shape=[jax.ShapeDtypeStruct((EPAD, CW), jnp.float32)] * 5,
    )(gl, gr, att_d.reshape(1, H))


def _tc_finish(na, nb, d0, d1, b):
    bm = 1024

    def body(na_ref, nb_ref, d0_ref, d1_ref, b_ref, h_ref):
        den = d0_ref[...][:, 0:1] + d1_ref[...][:, 0:1] + 1e-16
        h = jnp.tanh((na_ref[...] + nb_ref[...]) / den + b_ref[...])
        rid = (lax.broadcasted_iota(jnp.int32, (bm, 1), 0)
               + pl.program_id(0) * bm)
        h_ref[...] = jnp.where(rid < N, h, 0.0)

    return pl.pallas_call(
        body,
        grid=(NPAD // bm,),
        in_specs=[
            pl.BlockSpec((bm, H), lambda i: (i, 0)),
            pl.BlockSpec((bm, H), lambda i: (i, 0)),
            pl.BlockSpec((bm, CW), lambda i: (i, 0)),
            pl.BlockSpec((bm, CW), lambda i: (i, 0)),
            pl.BlockSpec((1, H), lambda i: (0, 0)),
        ],
        out_specs=pl.BlockSpec((bm, H), lambda i: (i, 0)),
        out_shape=jax.ShapeDtypeStruct((NPAD, H), jnp.float32),
    )(na, nb, d0, d1, b.reshape(1, H))


def _tc_head(h, w4, b4, w5, b5, w6, b6):
    bm = 1024

    def body(h_ref, w4_ref, b4_ref, w5_ref, b5_ref, w6_ref, b6_ref, o_ref):
        t = jnp.dot(h_ref[...], w4_ref[...],
                    preferred_element_type=jnp.float32) + b4_ref[...]
        t = jnp.maximum(t, 0.01 * t)
        t = jnp.dot(t, w5_ref[...],
                    preferred_element_type=jnp.float32) + b5_ref[...]
        t = jnp.maximum(t, 0.01 * t)
        o_ref[...] = jnp.dot(t, w6_ref[...],
                             preferred_element_type=jnp.float32) + b6_ref[...]

    return pl.pallas_call(
        body,
        grid=(NPAD // bm,),
        in_specs=[
            pl.BlockSpec((bm, H), lambda i: (i, 0)),
            pl.BlockSpec((H, H), lambda i: (0, 0)),
            pl.BlockSpec((1, H), lambda i: (0, 0)),
            pl.BlockSpec((H, H), lambda i: (0, 0)),
            pl.BlockSpec((1, H), lambda i: (0, 0)),
            pl.BlockSpec((H, 1), lambda i: (0, 0)),
            pl.BlockSpec((1, 1), lambda i: (0, 0)),
        ],
        out_specs=pl.BlockSpec((bm, 1), lambda i: (i, 0)),
        out_shape=jax.ShapeDtypeStruct((NPAD, 1), jnp.float32),
    )(h, w4, b4.reshape(1, H), w5, b5.reshape(1, H),
      w6.reshape(H, 1), b6.reshape(1, 1))


def _tc_softmax(logits, hits):
    def body(lg_ref, h_ref, o_ref):
        lg = lg_ref[...]
        hv = h_ref[...]
        nbr = hv[:NPAD, 0:1] + hv[NPAD:, 0:1]
        rows = lax.broadcasted_iota(jnp.int32, (NPAD, 1), 0)
        valid = rows < N
        lm = jnp.where(valid & (nbr > 0), lg,
                       jnp.where(valid, jnp.float32(-1e9),
                                 jnp.float32(-1e30)))
        m = jnp.max(lm)
        p = jnp.exp(lm - m)
        o_ref[...] = p / jnp.sum(p)

    return pl.pallas_call(
        body,
        out_shape=jax.ShapeDtypeStruct((NPAD, 1), jnp.float32),
    )(logits, hits)


def kernel(x, edge_index, current_vertex,
           Wl1, Wr1, att1, b1, Wl2, Wr2, att2, b2, Wl3, Wr3, att3, b3,
           W4, b4, W5, b5, W6, b6):
    f32 = jnp.float32
    xp = jnp.zeros((NPAD, DIN), f32).at[:N, :].set(x)
    loops = jnp.arange(N, dtype=jnp.int32)
    padi = jnp.full((EPAD - ET,), DUMMY, jnp.int32)
    src = jnp.concatenate([edge_index[0].astype(jnp.int32), loops, padi])
    dst = jnp.concatenate([edge_index[1].astype(jnp.int32), loops, padi])
    cv_row = jnp.full((1, 128), current_vertex, jnp.int32)

    flags = _tc_flags(src.reshape(EPAD // 128, 128), cv_row)
    flag128 = jnp.broadcast_to(flags.reshape(EPAD, 1), (EPAD, CW))
    hits = _sc_scatter_add(flag128, dst)

    perm = jnp.concatenate([jnp.arange(0, H, 2), jnp.arange(1, H, 2)])

    def pack(t):
        return lax.bitcast_convert_type(
            t.reshape(NPAD, HW, 2), jnp.int32)

    h = xp
    for li, (Wl, Wr, att, b) in enumerate((
            (Wl1, Wr1, att1, b1),
            (Wl2, Wr2, att2, b2),
            (Wl3, Wr3, att3, b3))):
        if li > 0:
            Wl, Wr = Wl[perm, :], Wr[perm, :]
        att_d = jnp.concatenate([att[0::2], att[1::2]])
        xl, xr = _tc_matmul2(h, Wl, Wr)
        gl = _sc_gather(pack(xl), src)
        gr = _sc_gather(pack(xr), dst)
        w128, p0, p1, p2, p3 = _tc_score(gl, gr, att_d)
        dsum = _sc_scatter_add(w128, dst)
        nums = [_sc_scatter_add(p, dst) for p in (p0, p1, p2, p3)]
        na = jnp.concatenate([n[:NPAD] for n in nums], axis=1)
        nb = jnp.concatenate([n[NPAD:] for n in nums], axis=1)
        h = _tc_finish(na, nb, dsum[:NPAD], dsum[NPAD:], b[perm])

    logits = _tc_head(h, W4[perm, :], b4, W5, b5, W6, b6)
    probs = _tc_softmax(logits, hits)
    return probs[:N, 0]
